# baseline (speedup 1.0000x reference)
import functools

import jax
import jax.numpy as jnp
from jax import lax
from jax.experimental import pallas as pl
from jax.experimental.pallas import tpu as pltpu
from jax.experimental.pallas import tpu_sc as plsc

_NUM_CORES = 1
_NUM_SUBCORES = 16
_NUM_WORKERS = _NUM_CORES * _NUM_SUBCORES
_CHUNK = 128


def _gather_body(n_chunks, values_hbm, t_hbm, out_hbm,
                 idx_v, out_v, sem_i, sem_g):
    wid = lax.axis_index("s") * _NUM_CORES + lax.axis_index("c")
    pltpu.async_copy(t_hbm.at[wid], idx_v, sem_i).wait()
    copies = [
        pltpu.async_copy(values_hbm.at[idx_v.at[j]], out_v.at[j], sem_g)
        for j in range(n_chunks)
    ]
    for cp in copies:
        cp.wait()
    pltpu.sync_copy(out_v, out_hbm.at[wid])


@jax.jit
def kernel(values, t):
    batch = t.shape[0]
    n_per_worker = batch // _NUM_WORKERS
    n_chunks = n_per_worker // _CHUNK
    t3 = t.reshape(_NUM_WORKERS, n_chunks, _CHUNK)

    mesh = plsc.VectorSubcoreMesh(
        core_axis_name="c", subcore_axis_name="s",
        num_cores=_NUM_CORES, num_subcores=_NUM_SUBCORES)
    gather = pl.kernel(
        functools.partial(_gather_body, n_chunks),
        out_type=jax.ShapeDtypeStruct((_NUM_WORKERS, n_chunks, _CHUNK),
                                      jnp.float32),
        mesh=mesh,
        scratch_types=[
            pltpu.VMEM((n_chunks, _CHUNK), jnp.int32),
            pltpu.VMEM((n_chunks, _CHUNK), jnp.float32),
            pltpu.SemaphoreType.DMA,
            pltpu.SemaphoreType.DMA,
        ],
        compiler_params=pltpu.CompilerParams(needs_layout_passes=False),
    )
    out = gather(values, t3)
    return out.reshape(batch, 1, 1, 1)


# speedup vs baseline: 1.5111x; 1.5111x over previous
import functools

import jax
import jax.numpy as jnp
from jax import lax
from jax.experimental import pallas as pl
from jax.experimental.pallas import tpu as pltpu
from jax.experimental.pallas import tpu_sc as plsc

_NUM_CORES = 1
_NUM_SUBCORES = 16
_NUM_WORKERS = _NUM_CORES * _NUM_SUBCORES
_LANES = 16


def _gather_body(n_per_worker, values_hbm, t_hbm, out_hbm,
                 table_v, idx_v, out_v, sem_t, sem_i):
    wid = lax.axis_index("s") * _NUM_CORES + lax.axis_index("c")
    base = wid * n_per_worker
    cp_table = pltpu.async_copy(values_hbm, table_v, sem_t)
    cp_idx = pltpu.async_copy(t_hbm.at[pl.ds(base, n_per_worker)], idx_v, sem_i)
    cp_table.wait()
    cp_idx.wait()

    @plsc.parallel_loop(0, n_per_worker // _LANES, unroll=8)
    def _(i):
        idx = idx_v[pl.ds(i * _LANES, _LANES)]
        out_v[pl.ds(i * _LANES, _LANES)] = plsc.load_gather(table_v, [idx])

    pltpu.sync_copy(out_v, out_hbm.at[pl.ds(base, n_per_worker)])


@jax.jit
def kernel(values, t):
    num_t = values.shape[0]
    batch = t.shape[0]
    n_per_worker = batch // _NUM_WORKERS

    mesh = plsc.VectorSubcoreMesh(
        core_axis_name="c", subcore_axis_name="s",
        num_cores=_NUM_CORES, num_subcores=_NUM_SUBCORES)
    gather = pl.kernel(
        functools.partial(_gather_body, n_per_worker),
        out_type=jax.ShapeDtypeStruct((batch,), jnp.float32),
        mesh=mesh,
        scratch_types=[
            pltpu.VMEM((num_t,), jnp.float32),
            pltpu.VMEM((n_per_worker,), jnp.int32),
            pltpu.VMEM((n_per_worker,), jnp.float32),
            pltpu.SemaphoreType.DMA,
            pltpu.SemaphoreType.DMA,
        ],
        compiler_params=pltpu.CompilerParams(
            needs_layout_passes=False),
    )
    out = gather(values, t)
    return out.reshape(batch, 1, 1, 1)
